# baseline (speedup 1.0000x reference)
import functools

import jax
import jax.numpy as jnp
from jax import lax
from jax.experimental import pallas as pl
from jax.experimental.pallas import tpu as pltpu
from jax.experimental.pallas import tpu_sc as plsc

H = 4
BQ = 256
BK = 256
BR = 512
SC_CHUNK = 128
SENTINEL = 0x3FFFFFFF


def _dotT(a, w):
    return lax.dot_general(a, w, (((1,), (1,)), ((), ())),
                           preferred_element_type=jnp.float32)


def _sc_scatter_partials(x_p, edges_p):
    NPAD, C = x_p.shape
    NCH_ALL = edges_p.shape[0]
    info = plsc.get_sparse_core_info()
    NC, NS = info.num_cores, info.num_subcores
    per_tile_pair = NCH_ALL // NS
    nch0 = max(2, (int(per_tile_pair * 0.975) // 2) * 2)
    nch1 = per_tile_pair - nch0
    rpt = NPAD // NS
    nz = rpt // SC_CHUNK

    mesh = plsc.VectorSubcoreMesh(core_axis_name="c", subcore_axis_name="s")

    @functools.partial(
        pl.kernel,
        out_type=jax.ShapeDtypeStruct((NC * NPAD, C), jnp.float32),
        mesh=mesh,
        scratch_types=[
            pltpu.VMEM((2, SC_CHUNK), jnp.int32),
            pltpu.VMEM((2, SC_CHUNK), jnp.int32),
            pltpu.VMEM((SC_CHUNK, C), jnp.float32),
            pltpu.VMEM((SC_CHUNK, C), jnp.float32),
            pltpu.VMEM_SHARED((NPAD, C), jnp.float32),
            pltpu.SemaphoreType.DMA,
            pltpu.SemaphoreType.DMA,
        ],
    )
    def sck(x_hbm, e_hbm, out_hbm, ib0, ib1, buf0, buf1, acc, sem0, sem1):
        c = lax.axis_index("c")
        s = lax.axis_index("s")
        nch = jnp.where(c == 0, nch0, nch1)
        base = jnp.where(c == 0, s * nch0, NS * nch0 + s * nch1)
        if True:
            z16 = jnp.zeros((16,), jnp.float32)

            def zrow(i, carry):
                for j in range(C // 16):
                    buf0[i, pl.ds(j * 16, 16)] = z16
                return carry

            lax.fori_loop(0, SC_CHUNK, zrow, 0)
            for z in range(nz):
                pltpu.sync_copy(
                    buf0, acc.at[pl.ds(s * rpt + z * SC_CHUNK, SC_CHUNK)])
            plsc.subcore_barrier()

            pltpu.sync_copy(e_hbm.at[base], ib0)
            pltpu.async_copy(x_hbm.at[ib0.at[0]], buf0, sem0)
            pltpu.sync_copy(e_hbm.at[base + 1], ib1)

            def pair(j, carry):
                g = j * 2
                pltpu.async_copy(x_hbm.at[ib1.at[0]], buf1, sem1)
                pltpu.make_async_copy(x_hbm.at[ib0.at[0]], buf0, sem0).wait()
                pltpu.sync_copy(buf0, acc.at[ib0.at[1]], add=True)

                @pl.when(g + 2 < nch)
                def _():
                    pltpu.sync_copy(e_hbm.at[base + g + 2], ib0)
                    pltpu.async_copy(x_hbm.at[ib0.at[0]], buf0, sem0)

                pltpu.make_async_copy(x_hbm.at[ib1.at[0]], buf1, sem1).wait()
                pltpu.sync_copy(buf1, acc.at[ib1.at[1]], add=True)

                @pl.when(g + 3 < nch)
                def _():
                    pltpu.sync_copy(e_hbm.at[base + g + 3], ib1)

                return carry

            lax.fori_loop(0, nch // 2, pair, 0)
            plsc.subcore_barrier()
            pltpu.sync_copy(acc.at[pl.ds(s * rpt, rpt)],
                            out_hbm.at[pl.ds(c * NPAD + s * rpt, rpt)])

    return sck(x_p, edges_p).reshape(NC, NPAD, C)


def _tck_qkv(x_p, Wq, bq):
    NPAD, C = x_p.shape

    def body(x_ref, wq_ref, bq_ref, qkv_ref):
        qkv_ref[...] = _dotT(x_ref[...], wq_ref[...]) + bq_ref[...]

    full = lambda r: (0, 0)
    return pl.pallas_call(
        body,
        grid=(NPAD // BR,),
        in_specs=[
            pl.BlockSpec((BR, C), lambda r: (r, 0)),
            pl.BlockSpec((3 * C, C), full),
            pl.BlockSpec((1, 3 * C), full),
        ],
        out_specs=pl.BlockSpec((BR, 3 * C), lambda r: (r, 0)),
        out_shape=jax.ShapeDtypeStruct((NPAD, 3 * C), jnp.float32),
    )(x_p, Wq, bq)


def _tck_gin(x_p, agg, mask_col, W1l, b1l, W2l, b2l):
    NPAD, C = x_p.shape

    def body(x_ref, agg_ref, m_ref, w1_ref, b1_ref, w2_ref, b2_ref,
             hpre_ref, s1_ref):
        r = pl.program_id(0)
        xb = x_ref[...]
        hin = agg_ref[0] + agg_ref[1] + xb
        t = jnp.maximum(_dotT(hin, w1_ref[...]) + b1_ref[...], 0.0)
        h = _dotT(t, w2_ref[...]) + b2_ref[...] + xb
        hpre_ref[...] = h
        hm = h * m_ref[...]
        part = jnp.concatenate(
            [jnp.sum(hm, 0, keepdims=True),
             jnp.sum(h * hm, 0, keepdims=True),
             jnp.zeros((6, C), jnp.float32)], axis=0)

        @pl.when(r == 0)
        def _():
            s1_ref[...] = jnp.zeros_like(s1_ref)

        s1_ref[...] += part

    grid = (NPAD // BR,)
    full = lambda r: (0, 0)
    return pl.pallas_call(
        body,
        grid=grid,
        in_specs=[
            pl.BlockSpec((BR, C), lambda r: (r, 0)),
            pl.BlockSpec((2, BR, C), lambda r: (0, r, 0)),
            pl.BlockSpec((BR, 1), lambda r: (r, 0)),
            pl.BlockSpec((C, C), full),
            pl.BlockSpec((1, C), full),
            pl.BlockSpec((C, C), full),
            pl.BlockSpec((1, C), full),
        ],
        out_specs=[
            pl.BlockSpec((BR, C), lambda r: (r, 0)),
            pl.BlockSpec((8, C), full),
        ],
        out_shape=[
            jax.ShapeDtypeStruct((NPAD, C), jnp.float32),
            jax.ShapeDtypeStruct((8, C), jnp.float32),
        ],
    )(x_p, agg, mask_col, W1l, b1l, W2l, b2l)


def _tck_attn(kr, q_a, k_a, v_a, bcol, btile, x_p, mask_col, Wo, bo):
    NPAD, C = q_a.shape
    NKB = NPAD // BK
    dh = C // H
    scale = 1.0 / (dh ** 0.5)

    def body(kr_ref, q_ref, k_ref, v_ref, bq_ref, bt_ref, x_ref, m_ref,
             wo_ref, bo_ref, h2_ref, s2_ref):
        qb = pl.program_id(0)
        t0 = kr_ref[0, qb]
        t1 = kr_ref[1, qb]
        q = q_ref[...] * scale
        bq = bq_ref[...]
        lane = lax.broadcasted_iota(jnp.int32, (1, C), 1)
        hmask = [(lane // dh == hh).astype(jnp.float32) for hh in range(H)]

        def step(t, carry):
            ms, ls, acc = carry
            off = pl.multiple_of(t * BK, BK)
            kt = k_ref[pl.ds(off, BK), :]
            vt = v_ref[pl.ds(off, BK), :]
            bk = bt_ref[pl.ds(t, 1)].reshape(1, BK)
            valid = bq == bk
            ms2, ls2, pv, af = [], [], None, None
            for hh in range(H):
                logits = lax.dot_general(
                    q, kt * hmask[hh], (((1,), (1,)), ((), ())),
                    preferred_element_type=jnp.float32)
                logits = jnp.where(valid, logits, -1e30)
                mn = jnp.maximum(ms[hh],
                                 jnp.max(logits, axis=1, keepdims=True))
                p = jnp.exp(logits - mn)
                alpha = jnp.exp(ms[hh] - mn)
                ms2.append(mn)
                ls2.append(ls[hh] * alpha + jnp.sum(p, axis=1, keepdims=True))
                pvh = lax.dot_general(p, vt * hmask[hh],
                                      (((1,), (0,)), ((), ())),
                                      preferred_element_type=jnp.float32)
                pv = pvh if pv is None else pv + pvh
                a = alpha * hmask[hh]
                af = a if af is None else af + a
            return tuple(ms2), tuple(ls2), acc * af + pv

        m0 = tuple(jnp.full((BQ, 1), -1e30, jnp.float32) for _ in range(H))
        l0 = tuple(jnp.zeros((BQ, 1), jnp.float32) for _ in range(H))
        a0 = jnp.zeros((BQ, C), jnp.float32)
        ms, ls, acc = lax.fori_loop(t0, t1, step, (m0, l0, a0))
        linv = sum((1.0 / jnp.maximum(ls[hh], 1e-30)) * hmask[hh]
                   for hh in range(H))
        o = acc * linv
        h2 = _dotT(o, wo_ref[...]) + bo_ref[...] + x_ref[...]
        h2_ref[...] = h2
        hm = h2 * m_ref[...]
        part = jnp.concatenate(
            [jnp.sum(hm, 0, keepdims=True),
             jnp.sum(h2 * hm, 0, keepdims=True),
             jnp.zeros((6, C), jnp.float32)], axis=0)

        @pl.when(qb == 0)
        def _():
            s2_ref[...] = jnp.zeros_like(s2_ref)

        s2_ref[...] += part

    grid = (NPAD // BQ,)
    full = lambda qb: (0, 0)
    return pl.pallas_call(
        body,
        grid=grid,
        in_specs=[
            pl.BlockSpec(memory_space=pltpu.SMEM),
            pl.BlockSpec((BQ, C), lambda qb: (qb, 0)),
            pl.BlockSpec((NPAD, C), full),
            pl.BlockSpec((NPAD, C), full),
            pl.BlockSpec((BQ, 1), lambda qb: (qb, 0)),
            pl.BlockSpec((NKB, 1, BK), lambda qb: (0, 0, 0)),
            pl.BlockSpec((BQ, C), lambda qb: (qb, 0)),
            pl.BlockSpec((BQ, 1), lambda qb: (qb, 0)),
            pl.BlockSpec((C, C), full),
            pl.BlockSpec((1, C), full),
        ],
        out_specs=[
            pl.BlockSpec((BQ, C), lambda qb: (qb, 0)),
            pl.BlockSpec((8, C), full),
        ],
        out_shape=[
            jax.ShapeDtypeStruct((NPAD, C), jnp.float32),
            jax.ShapeDtypeStruct((8, C), jnp.float32),
        ],
    )(kr, q_a, k_a, v_a, bcol, btile, x_p, mask_col, Wo, bo)


def _tck_ffn(hpre, h2pre, mask_col, A1, B1, A2, B2, M1l, mb1l, M2l, mb2l):
    NPAD, C = hpre.shape

    def body(h_ref, h2_ref, m_ref, a1_ref, b1_ref, a2_ref, b2_ref,
             m1_ref, mb1_ref, m2_ref, mb2_ref, y_ref, s3_ref):
        r = pl.program_id(0)
        h1 = h_ref[...] * a1_ref[...] + b1_ref[...]
        h2 = h2_ref[...] * a2_ref[...] + b2_ref[...]
        out = h1 + h2
        t = jnp.maximum(_dotT(out, m1_ref[...]) + mb1_ref[...], 0.0)
        y = _dotT(t, m2_ref[...]) + mb2_ref[...] + out
        y_ref[...] = y
        ym = y * m_ref[...]
        part = jnp.concatenate(
            [jnp.sum(ym, 0, keepdims=True),
             jnp.sum(y * ym, 0, keepdims=True),
             jnp.zeros((6, C), jnp.float32)], axis=0)

        @pl.when(r == 0)
        def _():
            s3_ref[...] = jnp.zeros_like(s3_ref)

        s3_ref[...] += part

    grid = (NPAD // BR,)
    full = lambda r: (0, 0)
    return pl.pallas_call(
        body,
        grid=grid,
        in_specs=[
            pl.BlockSpec((BR, C), lambda r: (r, 0)),
            pl.BlockSpec((BR, C), lambda r: (r, 0)),
            pl.BlockSpec((BR, 1), lambda r: (r, 0)),
            pl.BlockSpec((1, C), full),
            pl.BlockSpec((1, C), full),
            pl.BlockSpec((1, C), full),
            pl.BlockSpec((1, C), full),
            pl.BlockSpec((2 * C, C), full),
            pl.BlockSpec((1, 2 * C), full),
            pl.BlockSpec((C, 2 * C), full),
            pl.BlockSpec((1, C), full),
        ],
        out_specs=[
            pl.BlockSpec((BR, C), lambda r: (r, 0)),
            pl.BlockSpec((8, C), full),
        ],
        out_shape=[
            jax.ShapeDtypeStruct((NPAD, C), jnp.float32),
            jax.ShapeDtypeStruct((8, C), jnp.float32),
        ],
    )(hpre, h2pre, mask_col, A1, B1, A2, B2, M1l, mb1l, M2l, mb2l)


def _tck_bn_qkv(y, A3, B3, Wq, bq):
    NPAD, C = y.shape

    def body(y_ref, a_ref, b_ref, wq_ref, bq_ref, x_ref, qkv_ref):
        xn = y_ref[...] * a_ref[...] + b_ref[...]
        x_ref[...] = xn
        qkv_ref[...] = _dotT(xn, wq_ref[...]) + bq_ref[...]

    full = lambda r: (0, 0)
    return pl.pallas_call(
        body,
        grid=(NPAD // BR,),
        in_specs=[
            pl.BlockSpec((BR, C), lambda r: (r, 0)),
            pl.BlockSpec((1, C), full),
            pl.BlockSpec((1, C), full),
            pl.BlockSpec((3 * C, C), full),
            pl.BlockSpec((1, 3 * C), full),
        ],
        out_specs=[
            pl.BlockSpec((BR, C), lambda r: (r, 0)),
            pl.BlockSpec((BR, 3 * C), lambda r: (r, 0)),
        ],
        out_shape=[
            jax.ShapeDtypeStruct((NPAD, C), jnp.float32),
            jax.ShapeDtypeStruct((NPAD, 3 * C), jnp.float32),
        ],
    )(y, A3, B3, Wq, bq)


def _tck_bn_apply(y, A3, B3):
    NPAD, C = y.shape

    def body(y_ref, a_ref, b_ref, o_ref):
        o_ref[...] = y_ref[...] * a_ref[...] + b_ref[...]

    return pl.pallas_call(
        body,
        grid=(NPAD // BR,),
        in_specs=[
            pl.BlockSpec((BR, C), lambda r: (r, 0)),
            pl.BlockSpec((1, C), lambda r: (0, 0)),
            pl.BlockSpec((1, C), lambda r: (0, 0)),
        ],
        out_specs=pl.BlockSpec((BR, C), lambda r: (r, 0)),
        out_shape=jax.ShapeDtypeStruct((NPAD, C), jnp.float32),
    )(y, A3, B3)


def _bn_affine(stats, w, b, n):
    s = stats[0]
    s2 = stats[1]
    mean = s / n
    var = jnp.maximum(s2 / n - mean * mean, 0.0)
    a = w / jnp.sqrt(var + 1e-5)
    return (a.reshape(1, -1), (b - mean * a).reshape(1, -1))


def kernel(x, edge_index, batch, W1, b1, W2, b2, attn_in_w, attn_in_b,
           attn_out_w, attn_out_b, bn1_w, bn1_b, bn2_w, bn2_b, bn3_w, bn3_b,
           M1, mb1, M2, mb2):
    N, C = x.shape
    L = W1.shape[0]
    E = edge_index.shape[1]
    NPAD = (N // BQ + 1) * BQ
    NQB = NPAD // BQ
    NKB = NPAD // BK

    x_p = jnp.pad(x, ((0, NPAD - N), (0, 0)))
    batch_p = jnp.pad(batch, (0, NPAD - N), constant_values=SENTINEL)
    mask_col = (jnp.arange(NPAD) < N).astype(jnp.float32).reshape(NPAD, 1)
    bcol = batch_p.reshape(NPAD, 1)
    btile = batch_p.reshape(NKB, 1, BK)

    first = batch_p[0::BQ]
    last = batch_p[BQ - 1::BQ]
    t0 = (jnp.searchsorted(batch_p, first, side="left") // BK)
    t1 = ((jnp.searchsorted(batch_p, last, side="right") + BK - 1) // BK)
    kr = jnp.stack([t0, t1]).astype(jnp.int32)

    grp = 32 * SC_CHUNK * 2
    EP = ((E + grp - 1) // grp) * grp
    ei_p = jnp.concatenate(
        [edge_index.astype(jnp.int32),
         jnp.full((2, EP - E), N, jnp.int32)], axis=1)
    edges_p = ei_p.reshape(2, EP // SC_CHUNK, SC_CHUNK).transpose(1, 0, 2)

    nf = jnp.float32(N)
    for l in range(L):
        agg2 = _sc_scatter_partials(x_p, edges_p)
        qkv = _tck_qkv(x_p, attn_in_w[l], attn_in_b[l].reshape(1, 3 * C))
        q_a = qkv[:, :C]
        k_a = qkv[:, C:2 * C]
        v_a = qkv[:, 2 * C:]
        h2pre, s2 = _tck_attn(kr, q_a, k_a, v_a, bcol, btile, x_p, mask_col,
                              attn_out_w[l], attn_out_b[l].reshape(1, C))
        hpre, s1 = _tck_gin(x_p, agg2, mask_col, W1[l], b1[l].reshape(1, C),
                            W2[l], b2[l].reshape(1, C))
        A1, B1 = _bn_affine(s1, bn1_w[l], bn1_b[l], nf)
        A2, B2 = _bn_affine(s2, bn2_w[l], bn2_b[l], nf)
        y, s3 = _tck_ffn(hpre, h2pre, mask_col, A1, B1, A2, B2,
                         M1[l], mb1[l].reshape(1, 2 * C), M2[l],
                         mb2[l].reshape(1, C))
        A3, B3 = _bn_affine(s3, bn3_w[l], bn3_b[l], nf)
        x_p = _tck_bn_apply(y, A3, B3)

    return x_p[:N]


# speedup vs baseline: 1.0302x; 1.0302x over previous
import functools

import jax
import jax.numpy as jnp
from jax import lax
from jax.experimental import pallas as pl
from jax.experimental.pallas import tpu as pltpu
from jax.experimental.pallas import tpu_sc as plsc

H = 4
BQ = 256
BK = 256
BR = 512
SC_CHUNK = 128
SENTINEL = 0x3FFFFFFF


def _dotT(a, w):
    return lax.dot_general(a, w, (((1,), (1,)), ((), ())),
                           preferred_element_type=jnp.float32)


def _sc_scatter_partials(x_p, edges_p):
    NPAD, C = x_p.shape
    NCH_ALL = edges_p.shape[0]
    info = plsc.get_sparse_core_info()
    NC, NS = info.num_cores, info.num_subcores
    per_tile_pair = NCH_ALL // NS
    nch0 = max(2, (int(per_tile_pair * 0.95) // 2) * 2)
    nch1 = per_tile_pair - nch0
    rpt = NPAD // NS
    nz = rpt // SC_CHUNK

    mesh = plsc.VectorSubcoreMesh(core_axis_name="c", subcore_axis_name="s")

    @functools.partial(
        pl.kernel,
        out_type=jax.ShapeDtypeStruct((NC * NPAD, C), jnp.float32),
        mesh=mesh,
        scratch_types=[
            pltpu.VMEM((2, SC_CHUNK), jnp.int32),
            pltpu.VMEM((2, SC_CHUNK), jnp.int32),
            pltpu.VMEM((SC_CHUNK, C), jnp.float32),
            pltpu.VMEM((SC_CHUNK, C), jnp.float32),
            pltpu.VMEM_SHARED((NPAD, C), jnp.float32),
            pltpu.SemaphoreType.DMA,
            pltpu.SemaphoreType.DMA,
        ],
    )
    def sck(x_hbm, e_hbm, out_hbm, ib0, ib1, buf0, buf1, acc, sem0, sem1):
        c = lax.axis_index("c")
        s = lax.axis_index("s")
        nch = jnp.where(c == 0, nch0, nch1)
        base = jnp.where(c == 0, s * nch0, NS * nch0 + s * nch1)
        if True:
            z16 = jnp.zeros((16,), jnp.float32)

            def zrow(i, carry):
                for j in range(C // 16):
                    buf0[i, pl.ds(j * 16, 16)] = z16
                return carry

            lax.fori_loop(0, SC_CHUNK, zrow, 0)
            for z in range(nz):
                pltpu.sync_copy(
                    buf0, acc.at[pl.ds(s * rpt + z * SC_CHUNK, SC_CHUNK)])
            plsc.subcore_barrier()

            pltpu.sync_copy(e_hbm.at[base], ib0)
            pltpu.async_copy(x_hbm.at[ib0.at[0]], buf0, sem0)
            pltpu.sync_copy(e_hbm.at[base + 1], ib1)

            def pair(j, carry):
                g = j * 2
                pltpu.async_copy(x_hbm.at[ib1.at[0]], buf1, sem1)
                pltpu.make_async_copy(x_hbm.at[ib0.at[0]], buf0, sem0).wait()
                pltpu.sync_copy(buf0, acc.at[ib0.at[1]], add=True)

                @pl.when(g + 2 < nch)
                def _():
                    pltpu.sync_copy(e_hbm.at[base + g + 2], ib0)
                    pltpu.async_copy(x_hbm.at[ib0.at[0]], buf0, sem0)

                pltpu.make_async_copy(x_hbm.at[ib1.at[0]], buf1, sem1).wait()
                pltpu.sync_copy(buf1, acc.at[ib1.at[1]], add=True)

                @pl.when(g + 3 < nch)
                def _():
                    pltpu.sync_copy(e_hbm.at[base + g + 3], ib1)

                return carry

            lax.fori_loop(0, nch // 2, pair, 0)
            plsc.subcore_barrier()
            pltpu.sync_copy(acc.at[pl.ds(s * rpt, rpt)],
                            out_hbm.at[pl.ds(c * NPAD + s * rpt, rpt)])

    return sck(x_p, edges_p).reshape(NC, NPAD, C)


def _tck_qkv(x_p, Wq, bq):
    NPAD, C = x_p.shape

    def body(x_ref, wq_ref, bq_ref, qkv_ref):
        qkv_ref[...] = _dotT(x_ref[...], wq_ref[...]) + bq_ref[...]

    full = lambda r: (0, 0)
    return pl.pallas_call(
        body,
        grid=(NPAD // BR,),
        in_specs=[
            pl.BlockSpec((BR, C), lambda r: (r, 0)),
            pl.BlockSpec((3 * C, C), full),
            pl.BlockSpec((1, 3 * C), full),
        ],
        out_specs=pl.BlockSpec((BR, 3 * C), lambda r: (r, 0)),
        out_shape=jax.ShapeDtypeStruct((NPAD, 3 * C), jnp.float32),
    )(x_p, Wq, bq)


def _tck_gin(x_p, agg, mask_col, W1l, b1l, W2l, b2l):
    NPAD, C = x_p.shape

    def body(x_ref, agg_ref, m_ref, w1_ref, b1_ref, w2_ref, b2_ref,
             hpre_ref, s1_ref):
        r = pl.program_id(0)
        xb = x_ref[...]
        hin = agg_ref[0] + agg_ref[1] + xb
        t = jnp.maximum(_dotT(hin, w1_ref[...]) + b1_ref[...], 0.0)
        h = _dotT(t, w2_ref[...]) + b2_ref[...] + xb
        hpre_ref[...] = h
        hm = h * m_ref[...]
        part = jnp.concatenate(
            [jnp.sum(hm, 0, keepdims=True),
             jnp.sum(h * hm, 0, keepdims=True),
             jnp.zeros((6, C), jnp.float32)], axis=0)

        @pl.when(r == 0)
        def _():
            s1_ref[...] = jnp.zeros_like(s1_ref)

        s1_ref[...] += part

    grid = (NPAD // BR,)
    full = lambda r: (0, 0)
    return pl.pallas_call(
        body,
        grid=grid,
        in_specs=[
            pl.BlockSpec((BR, C), lambda r: (r, 0)),
            pl.BlockSpec((2, BR, C), lambda r: (0, r, 0)),
            pl.BlockSpec((BR, 1), lambda r: (r, 0)),
            pl.BlockSpec((C, C), full),
            pl.BlockSpec((1, C), full),
            pl.BlockSpec((C, C), full),
            pl.BlockSpec((1, C), full),
        ],
        out_specs=[
            pl.BlockSpec((BR, C), lambda r: (r, 0)),
            pl.BlockSpec((8, C), full),
        ],
        out_shape=[
            jax.ShapeDtypeStruct((NPAD, C), jnp.float32),
            jax.ShapeDtypeStruct((8, C), jnp.float32),
        ],
    )(x_p, agg, mask_col, W1l, b1l, W2l, b2l)


def _tck_attn(kr, q_a, k_a, v_a, bcol, btile, x_p, mask_col, Wo, bo):
    NPAD, C = q_a.shape
    NKB = NPAD // BK
    dh = C // H
    scale = 1.0 / (dh ** 0.5)

    def body(kr_ref, q_ref, k_ref, v_ref, bq_ref, bt_ref, x_ref, m_ref,
             wo_ref, bo_ref, h2_ref, s2_ref):
        qb = pl.program_id(0)
        t0 = kr_ref[0, qb]
        t1 = kr_ref[1, qb]
        q = q_ref[...] * scale
        bq = bq_ref[...]
        lane = lax.broadcasted_iota(jnp.int32, (1, C), 1)
        hmask = [(lane // dh == hh).astype(jnp.float32) for hh in range(H)]

        def step(t, carry):
            ms, ls, acc = carry
            off = pl.multiple_of(t * BK, BK)
            kt = k_ref[pl.ds(off, BK), :]
            vt = v_ref[pl.ds(off, BK), :]
            bk = bt_ref[pl.ds(t, 1)].reshape(1, BK)
            valid = bq == bk
            ms2, ls2, pv, af = [], [], None, None
            for hh in range(H):
                logits = lax.dot_general(
                    q, kt * hmask[hh], (((1,), (1,)), ((), ())),
                    preferred_element_type=jnp.float32)
                logits = jnp.where(valid, logits, -1e30)
                mn = jnp.maximum(ms[hh],
                                 jnp.max(logits, axis=1, keepdims=True))
                p = jnp.exp(logits - mn)
                alpha = jnp.exp(ms[hh] - mn)
                ms2.append(mn)
                ls2.append(ls[hh] * alpha + jnp.sum(p, axis=1, keepdims=True))
                pvh = lax.dot_general(p, vt * hmask[hh],
                                      (((1,), (0,)), ((), ())),
                                      preferred_element_type=jnp.float32)
                pv = pvh if pv is None else pv + pvh
                a = alpha * hmask[hh]
                af = a if af is None else af + a
            return tuple(ms2), tuple(ls2), acc * af + pv

        m0 = tuple(jnp.full((BQ, 1), -1e30, jnp.float32) for _ in range(H))
        l0 = tuple(jnp.zeros((BQ, 1), jnp.float32) for _ in range(H))
        a0 = jnp.zeros((BQ, C), jnp.float32)
        ms, ls, acc = lax.fori_loop(t0, t1, step, (m0, l0, a0))
        linv = sum((1.0 / jnp.maximum(ls[hh], 1e-30)) * hmask[hh]
                   for hh in range(H))
        o = acc * linv
        h2 = _dotT(o, wo_ref[...]) + bo_ref[...] + x_ref[...]
        h2_ref[...] = h2
        hm = h2 * m_ref[...]
        part = jnp.concatenate(
            [jnp.sum(hm, 0, keepdims=True),
             jnp.sum(h2 * hm, 0, keepdims=True),
             jnp.zeros((6, C), jnp.float32)], axis=0)

        @pl.when(qb == 0)
        def _():
            s2_ref[...] = jnp.zeros_like(s2_ref)

        s2_ref[...] += part

    grid = (NPAD // BQ,)
    full = lambda qb: (0, 0)
    return pl.pallas_call(
        body,
        grid=grid,
        in_specs=[
            pl.BlockSpec(memory_space=pltpu.SMEM),
            pl.BlockSpec((BQ, C), lambda qb: (qb, 0)),
            pl.BlockSpec((NPAD, C), full),
            pl.BlockSpec((NPAD, C), full),
            pl.BlockSpec((BQ, 1), lambda qb: (qb, 0)),
            pl.BlockSpec((NKB, 1, BK), lambda qb: (0, 0, 0)),
            pl.BlockSpec((BQ, C), lambda qb: (qb, 0)),
            pl.BlockSpec((BQ, 1), lambda qb: (qb, 0)),
            pl.BlockSpec((C, C), full),
            pl.BlockSpec((1, C), full),
        ],
        out_specs=[
            pl.BlockSpec((BQ, C), lambda qb: (qb, 0)),
            pl.BlockSpec((8, C), full),
        ],
        out_shape=[
            jax.ShapeDtypeStruct((NPAD, C), jnp.float32),
            jax.ShapeDtypeStruct((8, C), jnp.float32),
        ],
    )(kr, q_a, k_a, v_a, bcol, btile, x_p, mask_col, Wo, bo)


def _tck_ffn(hpre, h2pre, mask_col, A1, B1, A2, B2, M1l, mb1l, M2l, mb2l):
    NPAD, C = hpre.shape

    def body(h_ref, h2_ref, m_ref, a1_ref, b1_ref, a2_ref, b2_ref,
             m1_ref, mb1_ref, m2_ref, mb2_ref, y_ref, s3_ref):
        r = pl.program_id(0)
        h1 = h_ref[...] * a1_ref[...] + b1_ref[...]
        h2 = h2_ref[...] * a2_ref[...] + b2_ref[...]
        out = h1 + h2
        t = jnp.maximum(_dotT(out, m1_ref[...]) + mb1_ref[...], 0.0)
        y = _dotT(t, m2_ref[...]) + mb2_ref[...] + out
        y_ref[...] = y
        ym = y * m_ref[...]
        part = jnp.concatenate(
            [jnp.sum(ym, 0, keepdims=True),
             jnp.sum(y * ym, 0, keepdims=True),
             jnp.zeros((6, C), jnp.float32)], axis=0)

        @pl.when(r == 0)
        def _():
            s3_ref[...] = jnp.zeros_like(s3_ref)

        s3_ref[...] += part

    grid = (NPAD // BR,)
    full = lambda r: (0, 0)
    return pl.pallas_call(
        body,
        grid=grid,
        in_specs=[
            pl.BlockSpec((BR, C), lambda r: (r, 0)),
            pl.BlockSpec((BR, C), lambda r: (r, 0)),
            pl.BlockSpec((BR, 1), lambda r: (r, 0)),
            pl.BlockSpec((1, C), full),
            pl.BlockSpec((1, C), full),
            pl.BlockSpec((1, C), full),
            pl.BlockSpec((1, C), full),
            pl.BlockSpec((2 * C, C), full),
            pl.BlockSpec((1, 2 * C), full),
            pl.BlockSpec((C, 2 * C), full),
            pl.BlockSpec((1, C), full),
        ],
        out_specs=[
            pl.BlockSpec((BR, C), lambda r: (r, 0)),
            pl.BlockSpec((8, C), full),
        ],
        out_shape=[
            jax.ShapeDtypeStruct((NPAD, C), jnp.float32),
            jax.ShapeDtypeStruct((8, C), jnp.float32),
        ],
    )(hpre, h2pre, mask_col, A1, B1, A2, B2, M1l, mb1l, M2l, mb2l)


def _tck_bn_apply(y, A3, B3):
    NPAD, C = y.shape

    def body(y_ref, a_ref, b_ref, o_ref):
        o_ref[...] = y_ref[...] * a_ref[...] + b_ref[...]

    return pl.pallas_call(
        body,
        grid=(NPAD // BR,),
        in_specs=[
            pl.BlockSpec((BR, C), lambda r: (r, 0)),
            pl.BlockSpec((1, C), lambda r: (0, 0)),
            pl.BlockSpec((1, C), lambda r: (0, 0)),
        ],
        out_specs=pl.BlockSpec((BR, C), lambda r: (r, 0)),
        out_shape=jax.ShapeDtypeStruct((NPAD, C), jnp.float32),
    )(y, A3, B3)


def _bn_affine(stats, w, b, n):
    s = stats[0]
    s2 = stats[1]
    mean = s / n
    var = jnp.maximum(s2 / n - mean * mean, 0.0)
    a = w / jnp.sqrt(var + 1e-5)
    return (a.reshape(1, -1), (b - mean * a).reshape(1, -1))


def kernel(x, edge_index, batch, W1, b1, W2, b2, attn_in_w, attn_in_b,
           attn_out_w, attn_out_b, bn1_w, bn1_b, bn2_w, bn2_b, bn3_w, bn3_b,
           M1, mb1, M2, mb2):
    N, C = x.shape
    L = W1.shape[0]
    E = edge_index.shape[1]
    NPAD = (N // BQ + 1) * BQ
    NQB = NPAD // BQ
    NKB = NPAD // BK

    x_p = jnp.pad(x, ((0, NPAD - N), (0, 0)))
    batch_p = jnp.pad(batch, (0, NPAD - N), constant_values=SENTINEL)
    mask_col = (jnp.arange(NPAD) < N).astype(jnp.float32).reshape(NPAD, 1)
    bcol = batch_p.reshape(NPAD, 1)
    btile = batch_p.reshape(NKB, 1, BK)

    first = batch_p[0::BQ]
    last = batch_p[BQ - 1::BQ]
    t0 = (jnp.searchsorted(batch_p, first, side="left") // BK)
    t1 = ((jnp.searchsorted(batch_p, last, side="right") + BK - 1) // BK)
    kr = jnp.stack([t0, t1]).astype(jnp.int32)

    grp = 32 * SC_CHUNK * 2
    EP = ((E + grp - 1) // grp) * grp
    ei_p = jnp.concatenate(
        [edge_index.astype(jnp.int32),
         jnp.full((2, EP - E), N, jnp.int32)], axis=1)
    edges_p = ei_p.reshape(2, EP // SC_CHUNK, SC_CHUNK).transpose(1, 0, 2)

    nf = jnp.float32(N)
    for l in range(L):
        agg2 = _sc_scatter_partials(x_p, edges_p)
        qkv = _tck_qkv(x_p, attn_in_w[l], attn_in_b[l].reshape(1, 3 * C))
        q_a = qkv[:, :C]
        k_a = qkv[:, C:2 * C]
        v_a = qkv[:, 2 * C:]
        h2pre, s2 = _tck_attn(kr, q_a, k_a, v_a, bcol, btile, x_p, mask_col,
                              attn_out_w[l], attn_out_b[l].reshape(1, C))
        hpre, s1 = _tck_gin(x_p, agg2, mask_col, W1[l], b1[l].reshape(1, C),
                            W2[l], b2[l].reshape(1, C))
        A1, B1 = _bn_affine(s1, bn1_w[l], bn1_b[l], nf)
        A2, B2 = _bn_affine(s2, bn2_w[l], bn2_b[l], nf)
        y, s3 = _tck_ffn(hpre, h2pre, mask_col, A1, B1, A2, B2,
                         M1[l], mb1[l].reshape(1, 2 * C), M2[l],
                         mb2[l].reshape(1, C))
        A3, B3 = _bn_affine(s3, bn3_w[l], bn3_b[l], nf)
        x_p = _tck_bn_apply(y, A3, B3)

    return x_p[:N]
